# baseline (speedup 1.0000x reference)
import functools

import jax
import jax.numpy as jnp
from jax import lax
from jax.experimental import pallas as pl
from jax.experimental.pallas import tpu as pltpu

HID = 128
HEADS = 4
NUM_HERBS = 4096
NUM_SYMPTOMS = 2048
NUM_INGREDIENTS = 8192
NUM_HYPEREDGES = 2048
BATCH = 256


_KBLK = 512


def _mlp_body(x_ref, fs_ref, fh_ref, w1_ref, b1_ref, w2_ref, b2_ref,
              out_ref, logits_ref):
    k = pl.program_id(0)

    @pl.when(k == 0)
    def _init():
        semb = jnp.dot(x_ref[...], fs_ref[...], preferred_element_type=jnp.float32)
        logits_ref[...] = lax.dot_general(
            semb, fh_ref[...], (((1,), (1,)), ((), ())),
            preferred_element_type=jnp.float32)
        out_ref[...] = jnp.broadcast_to(b2_ref[...], out_ref.shape)

    hid = jnp.maximum(
        jnp.dot(logits_ref[...], w1_ref[...], preferred_element_type=jnp.float32)
        + b1_ref[0], 0.0)
    out_ref[...] += jnp.dot(hid, w2_ref[...], preferred_element_type=jnp.float32)


def _predictor(x, final_sym, final_herb, W1, b1, W2, b2):
    nk = W1.shape[1] // _KBLK
    b1r = b1.reshape(nk, 1, _KBLK)
    b2r = b2.reshape(1, -1)
    return pl.pallas_call(
        _mlp_body,
        grid=(nk,),
        in_specs=[
            pl.BlockSpec(x.shape, lambda k: (0, 0)),
            pl.BlockSpec(final_sym.shape, lambda k: (0, 0)),
            pl.BlockSpec(final_herb.shape, lambda k: (0, 0)),
            pl.BlockSpec((W1.shape[0], _KBLK), lambda k: (0, k)),
            pl.BlockSpec((1, 1, _KBLK), lambda k: (k, 0, 0)),
            pl.BlockSpec((_KBLK, W2.shape[1]), lambda k: (k, 0)),
            pl.BlockSpec(b2r.shape, lambda k: (0, 0)),
        ],
        out_specs=pl.BlockSpec((x.shape[0], W2.shape[1]), lambda k: (0, 0)),
        out_shape=jax.ShapeDtypeStruct((x.shape[0], W2.shape[1]), jnp.float32),
        scratch_shapes=[pltpu.VMEM((x.shape[0], W1.shape[0]), jnp.float32)],
        compiler_params=pltpu.CompilerParams(
            dimension_semantics=("arbitrary",)),
    )(x, final_sym, final_herb, W1, b1r, W2, b2r)


def _gat(x, edge_index, heads, out_ch, concat, W, a_src, a_dst, b):
    N = x.shape[0]
    loops = jnp.arange(N, dtype=edge_index.dtype)
    src = jnp.concatenate([edge_index[0], loops])
    dst = jnp.concatenate([edge_index[1], loops])
    h = (x @ W).reshape(N, heads, out_ch)
    asrc = (h * a_src).sum(-1)
    adst = (h * a_dst).sum(-1)
    e = jax.nn.leaky_relu(asrc[src] + adst[dst], 0.2)
    m = jax.ops.segment_max(e, dst, num_segments=N)
    ex = jnp.exp(e - m[dst])
    den = jax.ops.segment_sum(ex, dst, num_segments=N)
    alpha = ex / (den[dst] + 1e-16)
    out = jax.ops.segment_sum(h[src] * alpha[:, :, None], dst, num_segments=N)
    out = out.reshape(N, heads * out_ch) if concat else out.mean(axis=1)
    return out + b


def _hyperconv(x, edge_index, W, b):
    node = edge_index[0]
    he = edge_index[1]
    N = x.shape[0]
    xw = x @ W
    ones_n = jnp.ones(node.shape[0], dtype=jnp.float32)
    D = jax.ops.segment_sum(ones_n, node, num_segments=N)
    Bdeg = jax.ops.segment_sum(ones_n, he, num_segments=NUM_HYPEREDGES)
    Dinv = jnp.where(D > 0, 1.0 / jnp.maximum(D, 1.0), 0.0)
    Binv = jnp.where(Bdeg > 0, 1.0 / jnp.maximum(Bdeg, 1.0), 0.0)
    msg = jax.ops.segment_sum(xw[node], he, num_segments=NUM_HYPEREDGES) * Binv[:, None]
    out = jax.ops.segment_sum(msg[he], node, num_segments=N) * Dinv[:, None]
    return out + b


def _scatter_mean(src, idx, dim_size):
    s = jax.ops.segment_sum(src, idx, num_segments=dim_size)
    c = jax.ops.segment_sum(jnp.ones(src.shape[0], dtype=src.dtype), idx, num_segments=dim_size)
    return s / jnp.maximum(c, 1.0)[:, None]


def kernel(x, herb_x, symptom_x, cross_x, hyper_x, params, herb_edge_index,
           symptom_edge_index, cross_edge_index, hyper_edge_index,
           hyper_edge_mapping):
    p = params
    hx = jax.nn.elu(_gat(herb_x, herb_edge_index, HEADS, HID, True, **p['herb_gat1']))
    hx = jax.nn.elu(_gat(hx, herb_edge_index, 1, HID, True, **p['herb_gat2']))
    sx = jax.nn.elu(_gat(symptom_x, symptom_edge_index, HEADS, HID, True, **p['sym_gat1']))
    sx = jax.nn.elu(_gat(sx, symptom_edge_index, 1, HID, True, **p['sym_gat2']))
    cx = jax.nn.elu(_gat(cross_x, cross_edge_index, HEADS, HID, True, **p['cross_gat1']))
    cx = jax.nn.elu(_gat(cx, cross_edge_index, 1, HID, True, **p['cross_gat2']))
    hy = jax.nn.relu(_hyperconv(hyper_x, hyper_edge_index, p['hyper1']['W'], p['hyper1']['b']))
    hy = _hyperconv(hy, hyper_edge_index, p['hyper2']['W'], p['hyper2']['b'])
    herb_from_hyper = _scatter_mean(hy, hyper_edge_mapping, NUM_HERBS)
    final_sym = sx + cx[:NUM_SYMPTOMS]
    final_herb = hx + cx[NUM_SYMPTOMS:] + herb_from_hyper
    return _predictor(x, final_sym, final_herb, p['pred_W1'], p['pred_b1'],
                      p['pred_W2'], p['pred_b2'])
